# Initial kernel scaffold; baseline (speedup 1.0000x reference)
import jax
import jax.numpy as jnp
from jax.experimental import pallas as pl


def kernel(x, edge_index, W1, b1, W2, b2):
    raise NotImplementedError("write your pallas kernel here")


# speedup vs baseline: 13.7801x; 13.7801x over previous
import functools

import jax
import jax.numpy as jnp
from jax import lax
from jax.experimental import pallas as pl
from jax.experimental.pallas import tpu as pltpu
from jax.experimental.pallas import tpu_sc as plsc

N_NODES = 10000
N_EDGES = 320000
D_FEAT = 128
NHID = 64

NC = 2
NS = 16
NW = NC * NS
EDGE_CHUNK = 80
EDGES_PER_TILE = N_EDGES // NW
N_CHUNKS = EDGES_PER_TILE // EDGE_CHUNK
N_PAD = 10240
ROWS_PER_SUB = N_PAD // NS
L = 16

DIDX_CHUNK = 400


def _sc_mesh():
  return plsc.VectorSubcoreMesh(core_axis_name="c", subcore_axis_name="s")


def _zero_rows(buf, nrows, width):
  z = jnp.zeros((L,), jnp.float32)

  def row(i, _):
    for j in range(width // L):
      buf[i, pl.ds(j * L, L)] = z
    return 0

  lax.fori_loop(0, nrows, row, 0)


def _deg_body(dst_hbm, out_hbm, didx, hist, obuf, tbuf, hists):
  cid = lax.axis_index("c")
  sid = lax.axis_index("s")
  wid = sid * NC + cid

  z = jnp.zeros((L,), jnp.float32)

  def zrow(i, _):
    hist[pl.ds(pl.multiple_of(i * L, L), L)] = z
    return 0

  lax.fori_loop(0, N_PAD // L, zrow, 0)

  def chunk(c, _):
    base = pl.multiple_of(wid * EDGES_PER_TILE + c * DIDX_CHUNK, 8)
    pltpu.sync_copy(dst_hbm.at[pl.ds(base, DIDX_CHUNK)], didx)

    def grp(k, _):
      d16 = didx[pl.ds(pl.multiple_of(k * L, L), L)]
      srt, _ = plsc.sort_key_val(d16, d16)
      iota = lax.iota(jnp.int32, L)
      prev = srt.at[jnp.maximum(iota - 1, 0)].get(mode="promise_in_bounds")
      nxt = srt.at[jnp.minimum(iota + 1, L - 1)].get(mode="promise_in_bounds")
      first = (iota == 0) | (srt != prev)
      last = (iota == L - 1) | (srt != nxt)
      pf = plsc.cummax(jnp.where(first, iota, 0))
      cnt = (iota - pf + 1).astype(jnp.float32)
      plsc.addupdate_scatter(hist, [srt], cnt, mask=last)
      return 0

    lax.fori_loop(0, DIDX_CHUNK // L, grp, 0)
    return 0

  lax.fori_loop(0, EDGES_PER_TILE // DIDX_CHUNK, chunk, 0)

  pltpu.sync_copy(hist, hists.at[sid])
  plsc.subcore_barrier()

  row0 = pl.multiple_of(sid * ROWS_PER_SUB, 128)

  def zobuf(i, _):
    obuf[pl.ds(pl.multiple_of(i * L, L), L)] = z
    return 0

  lax.fori_loop(0, ROWS_PER_SUB // L, zobuf, 0)
  for t in range(NS):
    pltpu.sync_copy(hists.at[t, pl.ds(row0, ROWS_PER_SUB)], tbuf)

    def addv(i, _):
      s = pl.ds(pl.multiple_of(i * L, L), L)
      obuf[s] = obuf[s] + tbuf[s]
      return 0

    lax.fori_loop(0, ROWS_PER_SUB // L, addv, 0)
  pltpu.sync_copy(obuf, out_hbm.at[cid, pl.ds(row0, ROWS_PER_SUB)])


def _deg_kernel(dst):
  f = pl.kernel(
      _deg_body,
      out_type=jax.ShapeDtypeStruct((NC, N_PAD), jnp.float32),
      mesh=_sc_mesh(),
      compiler_params=pltpu.CompilerParams(needs_layout_passes=False),
      scratch_types=[
          pltpu.VMEM((DIDX_CHUNK,), jnp.int32),
          pltpu.VMEM((N_PAD,), jnp.float32),
          pltpu.VMEM((ROWS_PER_SUB,), jnp.float32),
          pltpu.VMEM((ROWS_PER_SUB,), jnp.float32),
          pltpu.VMEM_SHARED((NS, N_PAD), jnp.float32),
      ],
  )
  return f(dst)


def _agg_body(y_hbm, src_hbm, dst_hbm, out_hbm, sidx, didx, rows, acc, sem):
  cid = lax.axis_index("c")
  sid = lax.axis_index("s")
  wid = sid * NC + cid

  _zero_rows(rows, EDGE_CHUNK, D_FEAT)
  for r in range(ROWS_PER_SUB // EDGE_CHUNK):
    pltpu.sync_copy(
        rows, acc.at[pl.ds(sid * ROWS_PER_SUB + r * EDGE_CHUNK, EDGE_CHUNK)])
  plsc.subcore_barrier()

  def step(i, _):
    base = pl.multiple_of(wid * EDGES_PER_TILE + i * EDGE_CHUNK, 8)
    pltpu.sync_copy(src_hbm.at[pl.ds(base, EDGE_CHUNK)], sidx)
    pltpu.sync_copy(dst_hbm.at[pl.ds(base, EDGE_CHUNK)], didx)
    pltpu.async_copy(y_hbm.at[sidx], rows, sem).wait()
    pltpu.sync_copy(rows, acc.at[didx], add=True)
    return 0

  lax.fori_loop(0, N_CHUNKS, step, 0)
  plsc.subcore_barrier()

  for r in range(ROWS_PER_SUB // EDGE_CHUNK):
    row0 = sid * ROWS_PER_SUB + r * EDGE_CHUNK
    pltpu.sync_copy(acc.at[pl.ds(row0, EDGE_CHUNK)], rows)
    pltpu.sync_copy(rows, out_hbm.at[cid, pl.ds(row0, EDGE_CHUNK)])


def _agg_kernel(y, src, dst):
  f = pl.kernel(
      _agg_body,
      out_type=jax.ShapeDtypeStruct((NC, N_PAD, D_FEAT), jnp.float32),
      mesh=_sc_mesh(),
      scratch_types=[
          pltpu.VMEM((EDGE_CHUNK,), jnp.int32),
          pltpu.VMEM((EDGE_CHUNK,), jnp.int32),
          pltpu.VMEM((EDGE_CHUNK, D_FEAT), jnp.float32),
          pltpu.VMEM_SHARED((N_PAD, D_FEAT), jnp.float32),
          pltpu.SemaphoreType.DMA,
      ],
  )
  return f(y, src, dst)


ROW_BLK = 1024


def _tc_b_body(deg_ref, x_ref, w_ref, dis_ref, y_ref):
  deg = deg_ref[0, :] + deg_ref[1, :] + 1.0
  dis = lax.rsqrt(deg)[:, None]
  dis_ref[...] = dis
  y_ref[...] = jnp.dot(x_ref[...], w_ref[...],
                       preferred_element_type=jnp.float32) * dis


def _tc_b(deg2, xp, W1p):
  return pl.pallas_call(
      _tc_b_body,
      grid=(N_PAD // ROW_BLK,),
      in_specs=[
          pl.BlockSpec((NC, ROW_BLK), lambda i: (0, i)),
          pl.BlockSpec((ROW_BLK, D_FEAT), lambda i: (i, 0)),
          pl.BlockSpec((D_FEAT, D_FEAT), lambda i: (0, 0)),
      ],
      out_specs=[
          pl.BlockSpec((ROW_BLK, 1), lambda i: (i, 0)),
          pl.BlockSpec((ROW_BLK, D_FEAT), lambda i: (i, 0)),
      ],
      out_shape=[
          jax.ShapeDtypeStruct((N_PAD, 1), jnp.float32),
          jax.ShapeDtypeStruct((N_PAD, D_FEAT), jnp.float32),
      ],
  )(deg2, xp, W1p)


def _tc_d_body(acc_ref, y1_ref, dis_ref, b1_ref, w_ref, y2_ref):
  agg = acc_ref[0] + acc_ref[1] + y1_ref[...]
  dis = dis_ref[...]
  h = jnp.maximum(agg[:, :NHID] * dis + b1_ref[...][None, :], 0.0)
  y2_ref[...] = jnp.dot(h, w_ref[...],
                        preferred_element_type=jnp.float32) * dis


def _tc_d(acc1, y1, dis, b1, W2):
  return pl.pallas_call(
      _tc_d_body,
      grid=(N_PAD // ROW_BLK,),
      in_specs=[
          pl.BlockSpec((NC, ROW_BLK, D_FEAT), lambda i: (0, i, 0)),
          pl.BlockSpec((ROW_BLK, D_FEAT), lambda i: (i, 0)),
          pl.BlockSpec((ROW_BLK, 1), lambda i: (i, 0)),
          pl.BlockSpec((NHID,), lambda i: (0,)),
          pl.BlockSpec((NHID, D_FEAT), lambda i: (0, 0)),
      ],
      out_specs=pl.BlockSpec((ROW_BLK, D_FEAT), lambda i: (i, 0)),
      out_shape=jax.ShapeDtypeStruct((N_PAD, D_FEAT), jnp.float32),
  )(acc1, y1, dis, b1, W2)


def _tc_f_body(acc_ref, y2_ref, dis_ref, b2_ref, out_ref):
  agg = acc_ref[0] + acc_ref[1] + y2_ref[...]
  out_ref[...] = agg * dis_ref[...] + b2_ref[...][None, :]


def _tc_f(acc2, y2, dis, b2):
  return pl.pallas_call(
      _tc_f_body,
      grid=(N_PAD // ROW_BLK,),
      in_specs=[
          pl.BlockSpec((NC, ROW_BLK, D_FEAT), lambda i: (0, i, 0)),
          pl.BlockSpec((ROW_BLK, D_FEAT), lambda i: (i, 0)),
          pl.BlockSpec((ROW_BLK, 1), lambda i: (i, 0)),
          pl.BlockSpec((D_FEAT,), lambda i: (0,)),
      ],
      out_specs=pl.BlockSpec((ROW_BLK, D_FEAT), lambda i: (i, 0)),
      out_shape=jax.ShapeDtypeStruct((N_PAD, D_FEAT), jnp.float32),
  )(acc2, y2, dis, b2)


@jax.jit
def kernel(x, edge_index, W1, b1, W2, b2):
  ei = edge_index.astype(jnp.int32)
  src = ei[0]
  dst = ei[1]

  xp = jnp.pad(x, ((0, N_PAD - N_NODES), (0, 0)))
  W1p = jnp.pad(W1, ((0, 0), (0, D_FEAT - NHID)))

  deg2 = _deg_kernel(dst)
  dis, y1 = _tc_b(deg2, xp, W1p)
  acc1 = _agg_kernel(y1, src, dst)
  y2 = _tc_d(acc1, y1, dis, b1, W2)
  acc2 = _agg_kernel(y2, src, dst)
  out = _tc_f(acc2, y2, dis, b2)
  return out[:N_NODES]


# speedup vs baseline: 29.2016x; 2.1191x over previous
import functools

import jax
import jax.numpy as jnp
from jax import lax
from jax.experimental import pallas as pl
from jax.experimental.pallas import tpu as pltpu
from jax.experimental.pallas import tpu_sc as plsc

N_NODES = 10000
N_EDGES = 320000
D_FEAT = 128
NHID = 64

NC = 2
NS = 16
NW = NC * NS
EDGE_CHUNK = 80
EDGES_PER_TILE = N_EDGES // NW
N_CHUNKS = EDGES_PER_TILE // EDGE_CHUNK
N_PAD = 10240
ROWS_PER_SUB = N_PAD // NS
L = 16

DIDX_CHUNK = 400


def _sc_mesh():
  return plsc.VectorSubcoreMesh(core_axis_name="c", subcore_axis_name="s")


def _zero_rows(buf, nrows, width):
  z = jnp.zeros((L,), jnp.float32)

  def row(i, _):
    for j in range(width // L):
      buf[i, pl.ds(j * L, L)] = z
    return 0

  lax.fori_loop(0, nrows, row, 0)


def _deg_body(dst_hbm, out_hbm, didx, hist, obuf, tbuf, hists):
  cid = lax.axis_index("c")
  sid = lax.axis_index("s")
  wid = sid * NC + cid

  z = jnp.zeros((L,), jnp.float32)

  def zrow(i, _):
    hist[pl.ds(pl.multiple_of(i * L, L), L)] = z
    return 0

  lax.fori_loop(0, N_PAD // L, zrow, 0)

  def chunk(c, _):
    base = pl.multiple_of(wid * EDGES_PER_TILE + c * DIDX_CHUNK, 8)
    pltpu.sync_copy(dst_hbm.at[pl.ds(base, DIDX_CHUNK)], didx)

    def grp(k, _):
      d16 = didx[pl.ds(pl.multiple_of(k * L, L), L)]
      srt, _ = plsc.sort_key_val(d16, d16)
      iota = lax.iota(jnp.int32, L)
      prev = srt.at[jnp.maximum(iota - 1, 0)].get(mode="promise_in_bounds")
      nxt = srt.at[jnp.minimum(iota + 1, L - 1)].get(mode="promise_in_bounds")
      first = (iota == 0) | (srt != prev)
      last = (iota == L - 1) | (srt != nxt)
      pf = plsc.cummax(jnp.where(first, iota, 0))
      cnt = (iota - pf + 1).astype(jnp.float32)
      plsc.addupdate_scatter(hist, [srt], cnt, mask=last)
      return 0

    lax.fori_loop(0, DIDX_CHUNK // L, grp, 0)
    return 0

  lax.fori_loop(0, EDGES_PER_TILE // DIDX_CHUNK, chunk, 0)

  pltpu.sync_copy(hist, hists.at[sid])
  plsc.subcore_barrier()

  row0 = pl.multiple_of(sid * ROWS_PER_SUB, 128)

  def zobuf(i, _):
    obuf[pl.ds(pl.multiple_of(i * L, L), L)] = z
    return 0

  lax.fori_loop(0, ROWS_PER_SUB // L, zobuf, 0)
  for t in range(NS):
    pltpu.sync_copy(hists.at[t, pl.ds(row0, ROWS_PER_SUB)], tbuf)

    def addv(i, _):
      s = pl.ds(pl.multiple_of(i * L, L), L)
      obuf[s] = obuf[s] + tbuf[s]
      return 0

    lax.fori_loop(0, ROWS_PER_SUB // L, addv, 0)
  pltpu.sync_copy(obuf, out_hbm.at[cid, pl.ds(row0, ROWS_PER_SUB)])


def _deg_kernel(dst):
  f = pl.kernel(
      _deg_body,
      out_type=jax.ShapeDtypeStruct((NC, N_PAD), jnp.float32),
      mesh=_sc_mesh(),
      compiler_params=pltpu.CompilerParams(needs_layout_passes=False),
      scratch_types=[
          pltpu.VMEM((DIDX_CHUNK,), jnp.int32),
          pltpu.VMEM((N_PAD,), jnp.float32),
          pltpu.VMEM((ROWS_PER_SUB,), jnp.float32),
          pltpu.VMEM((ROWS_PER_SUB,), jnp.float32),
          pltpu.VMEM_SHARED((NS, N_PAD), jnp.float32),
      ],
  )
  return f(dst)


def _agg_body(y_hbm, src_hbm, dst_hbm, out_hbm, didx, sidx, rows, acc,
              gsems, isems):
  cid = lax.axis_index("c")
  sid = lax.axis_index("s")
  wid = sid * NC + cid

  pltpu.sync_copy(dst_hbm.at[wid], didx)

  _zero_rows(rows[0], EDGE_CHUNK, D_FEAT)
  for r in range(ROWS_PER_SUB // EDGE_CHUNK):
    pltpu.sync_copy(
        rows[0],
        acc.at[pl.ds(sid * ROWS_PER_SUB + r * EDGE_CHUNK, EDGE_CHUNK)])
  plsc.subcore_barrier()

  def gather(slot, chunk):
    return pltpu.make_async_copy(y_hbm.at[sidx[slot]], rows[slot],
                                 gsems[slot])

  def load_sidx(slot, chunk):
    return pltpu.make_async_copy(src_hbm.at[wid].at[chunk], sidx[slot],
                                 isems[slot])

  load_sidx(0, 0).start()
  load_sidx(1, 1).start()

  def body(q, _):
    a = q * 2

    load_sidx(0, a).wait()
    gather(0, a).start()

    @pl.when(q > 0)
    def _():
      gather(1, a - 1).wait()
      pltpu.sync_copy(rows[1], acc.at[didx.at[a - 1]], add=True)

    @pl.when(a + 2 < N_CHUNKS)
    def _():
      load_sidx(0, a + 2).start()

    @pl.when(a + 1 < N_CHUNKS)
    def _():
      load_sidx(1, a + 1).wait()
      gather(1, a + 1).start()

    gather(0, a).wait()
    pltpu.sync_copy(rows[0], acc.at[didx.at[a]], add=True)

    @pl.when(a + 3 < N_CHUNKS)
    def _():
      load_sidx(1, a + 3).start()

    return 0

  lax.fori_loop(0, (N_CHUNKS + 1) // 2, body, 0)
  plsc.subcore_barrier()

  for r in range(ROWS_PER_SUB // EDGE_CHUNK):
    row0 = sid * ROWS_PER_SUB + r * EDGE_CHUNK
    pltpu.sync_copy(acc.at[pl.ds(row0, EDGE_CHUNK)], rows[0])
    pltpu.sync_copy(rows[0], out_hbm.at[cid, pl.ds(row0, EDGE_CHUNK)])


@functools.cache
def _agg_kernel_fn():
  return pl.kernel(
      _agg_body,
      out_type=jax.ShapeDtypeStruct((NC, N_PAD, D_FEAT), jnp.float32),
      mesh=_sc_mesh(),
      scratch_types=[
          pltpu.VMEM((N_CHUNKS, EDGE_CHUNK), jnp.int32),
          [pltpu.VMEM((EDGE_CHUNK,), jnp.int32)] * 2,
          [pltpu.VMEM((EDGE_CHUNK, D_FEAT), jnp.float32)] * 2,
          pltpu.VMEM_SHARED((N_PAD, D_FEAT), jnp.float32),
          [pltpu.SemaphoreType.DMA] * 2,
          [pltpu.SemaphoreType.DMA] * 2,
      ],
  )


def _agg_kernel(y, src, dst):
  return _agg_kernel_fn()(y, src, dst)


ROW_BLK = 1024


def _tc_b_body(deg_ref, x_ref, w_ref, dis_ref, y_ref):
  deg = deg_ref[0, :] + deg_ref[1, :] + 1.0
  dis = lax.rsqrt(deg)[:, None]
  dis_ref[...] = dis
  y_ref[...] = jnp.dot(x_ref[...], w_ref[...],
                       preferred_element_type=jnp.float32) * dis


def _tc_b(deg2, xp, W1p):
  return pl.pallas_call(
      _tc_b_body,
      grid=(N_PAD // ROW_BLK,),
      in_specs=[
          pl.BlockSpec((NC, ROW_BLK), lambda i: (0, i)),
          pl.BlockSpec((ROW_BLK, D_FEAT), lambda i: (i, 0)),
          pl.BlockSpec((D_FEAT, D_FEAT), lambda i: (0, 0)),
      ],
      out_specs=[
          pl.BlockSpec((ROW_BLK, 1), lambda i: (i, 0)),
          pl.BlockSpec((ROW_BLK, D_FEAT), lambda i: (i, 0)),
      ],
      out_shape=[
          jax.ShapeDtypeStruct((N_PAD, 1), jnp.float32),
          jax.ShapeDtypeStruct((N_PAD, D_FEAT), jnp.float32),
      ],
  )(deg2, xp, W1p)


def _tc_d_body(acc_ref, y1_ref, dis_ref, b1_ref, w_ref, y2_ref):
  agg = acc_ref[0] + acc_ref[1] + y1_ref[...]
  dis = dis_ref[...]
  h = jnp.maximum(agg[:, :NHID] * dis + b1_ref[...][None, :], 0.0)
  y2_ref[...] = jnp.dot(h, w_ref[...],
                        preferred_element_type=jnp.float32) * dis


def _tc_d(acc1, y1, dis, b1, W2):
  return pl.pallas_call(
      _tc_d_body,
      grid=(N_PAD // ROW_BLK,),
      in_specs=[
          pl.BlockSpec((NC, ROW_BLK, D_FEAT), lambda i: (0, i, 0)),
          pl.BlockSpec((ROW_BLK, D_FEAT), lambda i: (i, 0)),
          pl.BlockSpec((ROW_BLK, 1), lambda i: (i, 0)),
          pl.BlockSpec((NHID,), lambda i: (0,)),
          pl.BlockSpec((NHID, D_FEAT), lambda i: (0, 0)),
      ],
      out_specs=pl.BlockSpec((ROW_BLK, D_FEAT), lambda i: (i, 0)),
      out_shape=jax.ShapeDtypeStruct((N_PAD, D_FEAT), jnp.float32),
  )(acc1, y1, dis, b1, W2)


def _tc_f_body(acc_ref, y2_ref, dis_ref, b2_ref, out_ref):
  agg = acc_ref[0] + acc_ref[1] + y2_ref[...]
  out_ref[...] = agg * dis_ref[...] + b2_ref[...][None, :]


def _tc_f(acc2, y2, dis, b2):
  return pl.pallas_call(
      _tc_f_body,
      grid=(N_PAD // ROW_BLK,),
      in_specs=[
          pl.BlockSpec((NC, ROW_BLK, D_FEAT), lambda i: (0, i, 0)),
          pl.BlockSpec((ROW_BLK, D_FEAT), lambda i: (i, 0)),
          pl.BlockSpec((ROW_BLK, 1), lambda i: (i, 0)),
          pl.BlockSpec((D_FEAT,), lambda i: (0,)),
      ],
      out_specs=pl.BlockSpec((ROW_BLK, D_FEAT), lambda i: (i, 0)),
      out_shape=jax.ShapeDtypeStruct((N_PAD, D_FEAT), jnp.float32),
  )(acc2, y2, dis, b2)


@jax.jit
def kernel(x, edge_index, W1, b1, W2, b2):
  ei = edge_index.astype(jnp.int32)
  src = ei[0]
  dst = ei[1]
  src3 = src.reshape(NW, N_CHUNKS, EDGE_CHUNK)
  dst3 = dst.reshape(NW, N_CHUNKS, EDGE_CHUNK)

  xp = jnp.pad(x, ((0, N_PAD - N_NODES), (0, 0)))
  W1p = jnp.pad(W1, ((0, 0), (0, D_FEAT - NHID)))

  deg2 = _deg_kernel(dst)
  dis, y1 = _tc_b(deg2, xp, W1p)
  acc1 = _agg_kernel(y1, src3, dst3)
  y2 = _tc_d(acc1, y1, dis, b1, W2)
  acc2 = _agg_kernel(y2, src3, dst3)
  out = _tc_f(acc2, y2, dis, b2)
  return out[:N_NODES]
